# baseline (speedup 1.0000x reference)
import functools

import jax
import jax.numpy as jnp
from jax import lax
from jax.experimental import pallas as pl
from jax.experimental.pallas import tpu as pltpu
from jax.experimental.pallas import tpu_sc as plsc

_THRESHOLD = 0.8
_S = 4096
_C = 2048
_L = 16
_NC = 2
_NS = 16
_NW = _NC * _NS
_ROWS_PER_W = _S // _NW
_CH = 16
_NCHUNK = _ROWS_PER_W // _CH
_NPOLY = 12


def _sc_body(x_hbm, out_hbm, buf, part, sem0, sem1):
    cid = lax.axis_index("c")
    sid = lax.axis_index("s")
    wid = sid * _NC + cid
    base = wid * _ROWS_PER_W

    sems = (sem0, sem1)

    def start(k):
        return pltpu.async_copy(
            x_hbm.at[0, pl.ds(base + k * _CH, _CH), :],
            buf.at[k % 2],
            sems[k % 2],
        )

    row_iota = lax.iota(jnp.int32, _L)
    loss_acc = jnp.zeros((_L,), jnp.float32)
    count_acc = jnp.zeros((_L,), jnp.float32)

    pending = start(0)
    for k in range(_NCHUNK):
        b = k % 2
        nxt = start(k + 1) if k + 1 < _NCHUNK else None
        pending.wait()
        pending = nxt

        accs = tuple(buf[b, r, pl.ds(0, _L)] for r in range(_CH))

        def col_body(i, a):
            return tuple(
                jnp.maximum(a[r], buf[b, r, pl.ds(i * _L, _L)])
                for r in range(_CH)
            )

        accs = lax.fori_loop(1, _C // _L, col_body, accs)

        p_vec = jnp.zeros((_L,), jnp.float32)
        for r in range(_CH):
            m_r = jnp.max(accs[r])
            p_vec = jnp.where(row_iota == jnp.int32(r), m_r, p_vec)

        mask = p_vec > jnp.float32(_THRESHOLD)
        q = jnp.where(mask, jnp.float32(1.0) - p_vec, jnp.float32(0.0))
        poly = jnp.full((_L,), jnp.float32(1.0 / _NPOLY))
        for k_ in range(_NPOLY - 1, 0, -1):
            poly = poly * q + jnp.float32(1.0 / k_)
        neg_log_p = q * poly
        loss_acc = loss_acc + q * q * neg_log_p
        count_acc = count_acc + jnp.where(
            mask, jnp.float32(1.0), jnp.float32(0.0)
        )

    part[0, :] = loss_acc
    part[1, :] = count_acc
    pltpu.sync_copy(part.at[0], out_hbm.at[0, pl.ds(wid * _L, _L)])
    pltpu.sync_copy(part.at[1], out_hbm.at[1, pl.ds(wid * _L, _L)])


@functools.cache
def _make_sc_call():
    return pl.kernel(
        _sc_body,
        out_type=jax.ShapeDtypeStruct((2, _NW * _L), jnp.float32),
        mesh=plsc.VectorSubcoreMesh(
            core_axis_name="c", subcore_axis_name="s",
            num_cores=_NC, num_subcores=_NS,
        ),
        compiler_params=pltpu.CompilerParams(needs_layout_passes=False),
        scratch_types=[
            pltpu.VMEM((2, _CH, _C), jnp.float32),
            pltpu.VMEM((2, _L), jnp.float32),
            pltpu.SemaphoreType.DMA,
            pltpu.SemaphoreType.DMA,
        ],
    )


def _finish_body(p_ref, o_ref):
    loss_sum = jnp.sum(p_ref[0, :])
    count = jnp.sum(p_ref[1, :])
    val = loss_sum / jnp.maximum(count, jnp.float32(1.0))
    o_ref[...] = jnp.reshape(val, (1, 1))


def kernel(inputs):
    partials = _make_sc_call()(inputs)
    out = pl.pallas_call(
        _finish_body,
        out_shape=jax.ShapeDtypeStruct((1, 1), jnp.float32),
    )(partials)
    return out[0, 0]


# speedup vs baseline: 1.0083x; 1.0083x over previous
import functools

import jax
import jax.numpy as jnp
from jax import lax
from jax.experimental import pallas as pl
from jax.experimental.pallas import tpu as pltpu
from jax.experimental.pallas import tpu_sc as plsc

_THRESHOLD = 0.8
_S = 4096
_C = 2048
_L = 16
_NC = 2
_NS = 16
_NW = _NC * _NS
_ROWS_PER_W = _S // _NW
_CH = 16
_NCHUNK = _ROWS_PER_W // _CH
_NBUF = 3
_NPOLY = 12


def _sc_body(x_hbm, out0_hbm, out1_hbm, buf, part, sem0, sem1, sem2):
    cid = lax.axis_index("c")
    sid = lax.axis_index("s")
    wid = sid * _NC + cid
    base = wid * _ROWS_PER_W

    sems = (sem0, sem1, sem2)

    def start(k):
        return pltpu.async_copy(
            x_hbm.at[0, pl.ds(base + k * _CH, _CH), :],
            buf.at[k % _NBUF],
            sems[k % _NBUF],
        )

    row_iota = lax.iota(jnp.int32, _L)
    loss_acc = jnp.zeros((_L,), jnp.float32)
    count_acc = jnp.zeros((_L,), jnp.float32)

    pending = [start(0), start(1)]
    for k in range(_NCHUNK):
        b = k % _NBUF
        if k + 2 < _NCHUNK:
            pending.append(start(k + 2))
        pending.pop(0).wait()

        accs = tuple(
            jnp.maximum(buf[b, r, pl.ds(0, _L)], buf[b, r, pl.ds(_L, _L)])
            for r in range(_CH)
        )

        def col_body(i, a):
            base_c = i * (2 * _L)
            a = tuple(
                jnp.maximum(a[r], buf[b, r, pl.ds(base_c, _L)])
                for r in range(_CH)
            )
            return tuple(
                jnp.maximum(a[r], buf[b, r, pl.ds(base_c + _L, _L)])
                for r in range(_CH)
            )

        accs = lax.fori_loop(1, _C // (2 * _L), col_body, accs)

        p_vec = jnp.zeros((_L,), jnp.float32)
        for r in range(_CH):
            m_r = jnp.max(accs[r])
            p_vec = jnp.where(row_iota == jnp.int32(r), m_r, p_vec)

        mask = p_vec > jnp.float32(_THRESHOLD)
        q = jnp.where(mask, jnp.float32(1.0) - p_vec, jnp.float32(0.0))
        poly = jnp.full((_L,), jnp.float32(1.0 / _NPOLY))
        for k_ in range(_NPOLY - 1, 0, -1):
            poly = poly * q + jnp.float32(1.0 / k_)
        neg_log_p = q * poly
        loss_acc = loss_acc + q * q * neg_log_p
        count_acc = count_acc + jnp.where(
            mask, jnp.float32(1.0), jnp.float32(0.0)
        )

    part[0, :] = loss_acc
    part[1, :] = count_acc

    @pl.when(cid == 0)
    def _():
        pltpu.sync_copy(part.at[0], out0_hbm.at[0, pl.ds(sid * _L, _L)])
        pltpu.sync_copy(part.at[1], out0_hbm.at[1, pl.ds(sid * _L, _L)])

    @pl.when(cid == 1)
    def _():
        pltpu.sync_copy(part.at[0], out1_hbm.at[0, pl.ds(sid * _L, _L)])
        pltpu.sync_copy(part.at[1], out1_hbm.at[1, pl.ds(sid * _L, _L)])


@functools.cache
def _make_sc_call():
    return pl.kernel(
        _sc_body,
        out_type=(
            jax.ShapeDtypeStruct((2, _NS * _L), jnp.float32),
            jax.ShapeDtypeStruct((2, _NS * _L), jnp.float32),
        ),
        mesh=plsc.VectorSubcoreMesh(
            core_axis_name="c", subcore_axis_name="s",
            num_cores=_NC, num_subcores=_NS,
        ),
        compiler_params=pltpu.CompilerParams(needs_layout_passes=False),
        scratch_types=[
            pltpu.VMEM((_NBUF, _CH, _C), jnp.float32),
            pltpu.VMEM((2, _L), jnp.float32),
            pltpu.SemaphoreType.DMA,
            pltpu.SemaphoreType.DMA,
            pltpu.SemaphoreType.DMA,
        ],
    )


def _finish_body(p0_ref, p1_ref, o_ref):
    loss_sum = jnp.sum(p0_ref[0, :]) + jnp.sum(p1_ref[0, :])
    count = jnp.sum(p0_ref[1, :]) + jnp.sum(p1_ref[1, :])
    val = loss_sum / jnp.maximum(count, jnp.float32(1.0))
    o_ref[...] = jnp.reshape(val, (1, 1))


def kernel(inputs):
    part0, part1 = _make_sc_call()(inputs)
    out = pl.pallas_call(
        _finish_body,
        out_shape=jax.ShapeDtypeStruct((1, 1), jnp.float32),
    )(part0, part1)
    return out[0, 0]


# speedup vs baseline: 1.2051x; 1.1952x over previous
import functools

import jax
import jax.numpy as jnp
from jax import lax
from jax.experimental import pallas as pl
from jax.experimental.pallas import tpu as pltpu
from jax.experimental.pallas import tpu_sc as plsc

_THRESHOLD = 0.8
_S = 4096
_C = 2048
_L = 16
_NC = 2
_NS = 16
_NW = _NC * _NS

_R_SC = 1024
_ROWS_PER_W = _R_SC // _NW
_CH = 16
_NCHUNK = _ROWS_PER_W // _CH
_NBUF = 2
_NPOLY = 12

_R_TC = _S - _R_SC
_TC_BLOCK = 256
_TC_GRID = _R_TC // _TC_BLOCK


def _sc_body(x_hbm, out0_hbm, out1_hbm, buf, part, sem0, sem1):
    cid = lax.axis_index("c")
    sid = lax.axis_index("s")
    wid = sid * _NC + cid
    base = wid * _ROWS_PER_W

    sems = (sem0, sem1)

    def start(k):
        return pltpu.async_copy(
            x_hbm.at[0, pl.ds(base + k * _CH, _CH), :],
            buf.at[k % _NBUF],
            sems[k % _NBUF],
        )

    row_iota = lax.iota(jnp.int32, _L)
    loss_acc = jnp.zeros((_L,), jnp.float32)
    count_acc = jnp.zeros((_L,), jnp.float32)

    pending = [start(0), start(1)]
    for k in range(_NCHUNK):
        b = k % _NBUF
        if k + 2 < _NCHUNK:
            pending.append(start(k + 2))
        pending.pop(0).wait()

        accs = tuple(
            jnp.maximum(buf[b, r, pl.ds(0, _L)], buf[b, r, pl.ds(_L, _L)])
            for r in range(_CH)
        )

        def col_body(i, a):
            base_c = i * (2 * _L)
            a = tuple(
                jnp.maximum(a[r], buf[b, r, pl.ds(base_c, _L)])
                for r in range(_CH)
            )
            return tuple(
                jnp.maximum(a[r], buf[b, r, pl.ds(base_c + _L, _L)])
                for r in range(_CH)
            )

        accs = lax.fori_loop(1, _C // (2 * _L), col_body, accs)

        p_vec = jnp.zeros((_L,), jnp.float32)
        for r in range(_CH):
            m_r = jnp.max(accs[r])
            p_vec = jnp.where(row_iota == jnp.int32(r), m_r, p_vec)

        mask = p_vec > jnp.float32(_THRESHOLD)
        q = jnp.where(mask, jnp.float32(1.0) - p_vec, jnp.float32(0.0))
        poly = jnp.full((_L,), jnp.float32(1.0 / _NPOLY))
        for k_ in range(_NPOLY - 1, 0, -1):
            poly = poly * q + jnp.float32(1.0 / k_)
        neg_log_p = q * poly
        loss_acc = loss_acc + q * q * neg_log_p
        count_acc = count_acc + jnp.where(
            mask, jnp.float32(1.0), jnp.float32(0.0)
        )

    part[0, :] = loss_acc
    part[1, :] = count_acc

    @pl.when(cid == 0)
    def _():
        pltpu.sync_copy(part.at[0], out0_hbm.at[0, pl.ds(sid * _L, _L)])
        pltpu.sync_copy(part.at[1], out0_hbm.at[1, pl.ds(sid * _L, _L)])

    @pl.when(cid == 1)
    def _():
        pltpu.sync_copy(part.at[0], out1_hbm.at[0, pl.ds(sid * _L, _L)])
        pltpu.sync_copy(part.at[1], out1_hbm.at[1, pl.ds(sid * _L, _L)])


@functools.cache
def _make_sc_call():
    return pl.kernel(
        _sc_body,
        out_type=(
            jax.ShapeDtypeStruct((2, _NS * _L), jnp.float32),
            jax.ShapeDtypeStruct((2, _NS * _L), jnp.float32),
        ),
        mesh=plsc.VectorSubcoreMesh(
            core_axis_name="c", subcore_axis_name="s",
            num_cores=_NC, num_subcores=_NS,
        ),
        compiler_params=pltpu.CompilerParams(needs_layout_passes=False),
        scratch_types=[
            pltpu.VMEM((_NBUF, _CH, _C), jnp.float32),
            pltpu.VMEM((2, _L), jnp.float32),
            pltpu.SemaphoreType.DMA,
            pltpu.SemaphoreType.DMA,
        ],
    )


def _tc_rowmax_body(x_ref, o_ref):
    o_ref[0, 0, :] = jnp.max(x_ref[0], axis=1)


def _tc_rowmax(inputs):
    return pl.pallas_call(
        _tc_rowmax_body,
        grid=(_TC_GRID,),
        in_specs=[
            pl.BlockSpec(
                (1, _TC_BLOCK, _C),
                lambda i: (0, i + _R_SC // _TC_BLOCK, 0),
            )
        ],
        out_specs=pl.BlockSpec((1, 1, _TC_BLOCK), lambda i: (i, 0, 0)),
        out_shape=jax.ShapeDtypeStruct(
            (_TC_GRID, 1, _TC_BLOCK), jnp.float32
        ),
    )(inputs)


def _finish_body(p0_ref, p1_ref, ptc_ref, o_ref):
    loss_sum = jnp.sum(p0_ref[0, :]) + jnp.sum(p1_ref[0, :])
    count = jnp.sum(p0_ref[1, :]) + jnp.sum(p1_ref[1, :])
    p = ptc_ref[...]
    mask = p > jnp.float32(_THRESHOLD)
    safe_p = jnp.where(mask, p, jnp.float32(1.0))
    q = jnp.float32(1.0) - safe_p
    loss_tc = q * q * (-jnp.log(safe_p))
    loss_sum = loss_sum + jnp.sum(loss_tc)
    count = count + jnp.sum(mask.astype(jnp.float32))
    val = loss_sum / jnp.maximum(count, jnp.float32(1.0))
    o_ref[...] = jnp.reshape(val, (1, 1))


def kernel(inputs):
    part0, part1 = _make_sc_call()(inputs)
    p_tc = _tc_rowmax(inputs)
    out = pl.pallas_call(
        _finish_body,
        out_shape=jax.ShapeDtypeStruct((1, 1), jnp.float32),
    )(part0, part1, p_tc)
    return out[0, 0]
